# Initial kernel scaffold; baseline (speedup 1.0000x reference)
import jax
import jax.numpy as jnp
from jax.experimental import pallas as pl


def kernel(x, edge_index, batch, emb_W, emb_b, conv_W, conv_b, ln_g, ln_b, out_W):
    raise NotImplementedError("write your pallas kernel here")


# speedup vs baseline: 13.6995x; 13.6995x over previous
import functools

import jax
import jax.numpy as jnp
from jax import lax
from jax.experimental import pallas as pl
from jax.experimental.pallas import tpu as pltpu
from jax.experimental.pallas import tpu_sc as plsc

N = 10000
E = 320000
D = 128
G = 64

NC, NS = 2, 16
NW = NC * NS
EW = E // NW
CH = 128
NFULL = EW // CH
TAIL = EW - NFULL * CH
RS = 624
NT = N - NS * RS

_mesh = plsc.VectorSubcoreMesh(core_axis_name="c", subcore_axis_name="s")


@functools.partial(
    pl.kernel,
    out_type=jax.ShapeDtypeStruct((NC * N, 16), jnp.float32),
    mesh=_mesh,
    scratch_types=[
        pltpu.VMEM((CH,), jnp.int32),
        pltpu.VMEM((TAIL,), jnp.int32),
        pltpu.VMEM((CH, 16), jnp.float32),
        pltpu.VMEM_SHARED((N, 16), jnp.float32),
    ],
    compiler_params=pltpu.CompilerParams(use_tc_tiling_on_sc=False),
)
def _deg_kernel(dst_hbm, zeros_hbm, ones_hbm, out_hbm, idx_v, idxt_v, ones_v,
                acc):
    c = lax.axis_index("c")
    s = lax.axis_index("s")
    base_w = (c * NS + s) * EW
    row0 = s * RS
    pltpu.sync_copy(zeros_hbm.at[pl.ds(row0, RS)], acc.at[pl.ds(row0, RS)])

    @pl.when(s == 0)
    def _():
        pltpu.sync_copy(zeros_hbm.at[pl.ds(NS * RS, NT)],
                        acc.at[pl.ds(NS * RS, NT)])

    pltpu.sync_copy(ones_hbm, ones_v)
    plsc.subcore_barrier()

    def body(k, carry):
        base = base_w + k * CH
        pltpu.sync_copy(dst_hbm.at[pl.ds(base, CH)], idx_v)
        pltpu.sync_copy(ones_v, acc.at[idx_v], add=True)
        return carry

    lax.fori_loop(0, NFULL, body, 0)
    pltpu.sync_copy(dst_hbm.at[pl.ds(base_w + NFULL * CH, TAIL)], idxt_v)
    pltpu.sync_copy(ones_v.at[pl.ds(0, TAIL)], acc.at[idxt_v], add=True)
    plsc.subcore_barrier()
    pltpu.sync_copy(acc.at[pl.ds(row0, RS)],
                    out_hbm.at[pl.ds(c * N + row0, RS)])

    @pl.when(s == 0)
    def _():
        pltpu.sync_copy(acc.at[pl.ds(NS * RS, NT)],
                        out_hbm.at[pl.ds(c * N + NS * RS, NT)])


@functools.partial(
    pl.kernel,
    out_type=jax.ShapeDtypeStruct((NC * N, D), jnp.float32),
    mesh=_mesh,
    scratch_types=[
        pltpu.VMEM((CH,), jnp.int32),
        pltpu.VMEM((CH,), jnp.int32),
        pltpu.VMEM((TAIL,), jnp.int32),
        pltpu.VMEM((TAIL,), jnp.int32),
        pltpu.VMEM((CH, D), jnp.float32),
        pltpu.VMEM((TAIL, D), jnp.float32),
        pltpu.VMEM_SHARED((N, D), jnp.float32),
        pltpu.SemaphoreType.DMA,
    ],
)
def _agg_kernel(hws_hbm, src_hbm, dst_hbm, zeros_hbm, out_hbm,
                sidx, didx, sidx_t, didx_t, rows, rows_t, acc, sem):
    c = lax.axis_index("c")
    s = lax.axis_index("s")
    base_w = (c * NS + s) * EW
    row0 = s * RS
    pltpu.sync_copy(zeros_hbm.at[pl.ds(row0, RS)], acc.at[pl.ds(row0, RS)])

    @pl.when(s == 0)
    def _():
        pltpu.sync_copy(zeros_hbm.at[pl.ds(NS * RS, NT)],
                        acc.at[pl.ds(NS * RS, NT)])

    plsc.subcore_barrier()

    def body(k, carry):
        base = base_w + k * CH
        pltpu.sync_copy(src_hbm.at[pl.ds(base, CH)], sidx)
        pltpu.sync_copy(dst_hbm.at[pl.ds(base, CH)], didx)
        pltpu.async_copy(hws_hbm.at[sidx], rows, sem).wait()
        pltpu.sync_copy(rows, acc.at[didx], add=True)
        return carry

    lax.fori_loop(0, NFULL, body, 0)
    base = base_w + NFULL * CH
    pltpu.sync_copy(src_hbm.at[pl.ds(base, TAIL)], sidx_t)
    pltpu.sync_copy(dst_hbm.at[pl.ds(base, TAIL)], didx_t)
    pltpu.async_copy(hws_hbm.at[sidx_t], rows_t, sem).wait()
    pltpu.sync_copy(rows_t, acc.at[didx_t], add=True)
    plsc.subcore_barrier()
    pltpu.sync_copy(acc.at[pl.ds(row0, RS)],
                    out_hbm.at[pl.ds(c * N + row0, RS)])

    @pl.when(s == 0)
    def _():
        pltpu.sync_copy(acc.at[pl.ds(NS * RS, NT)],
                        out_hbm.at[pl.ds(c * N + NS * RS, NT)])


R = 1000


def _pro_body(x_ref, ew_ref, eb_ref, w0_ref, degp_ref, h_ref, hws_ref, dis_ref):
    deg = degp_ref[0, :, 0:1] + degp_ref[1, :, 0:1] + 1.0
    dis = lax.rsqrt(deg)
    h = jnp.dot(x_ref[...], ew_ref[...], preferred_element_type=jnp.float32)
    h = h + eb_ref[...]
    hw = jnp.dot(h, w0_ref[...], preferred_element_type=jnp.float32)
    h_ref[...] = h
    hws_ref[...] = hw * dis
    dis_ref[...] = dis


_pro = pl.pallas_call(
    _pro_body,
    grid=(N // R,),
    in_specs=[
        pl.BlockSpec((R, D), lambda i: (i, 0)),
        pl.BlockSpec((D, D), lambda i: (0, 0)),
        pl.BlockSpec((1, D), lambda i: (0, 0)),
        pl.BlockSpec((D, D), lambda i: (0, 0)),
        pl.BlockSpec((NC, R, 16), lambda i: (0, i, 0)),
    ],
    out_specs=[
        pl.BlockSpec((R, D), lambda i: (i, 0)),
        pl.BlockSpec((R, D), lambda i: (i, 0)),
        pl.BlockSpec((R, 1), lambda i: (i, 0)),
    ],
    out_shape=[
        jax.ShapeDtypeStruct((N, D), jnp.float32),
        jax.ShapeDtypeStruct((N, D), jnp.float32),
        jax.ShapeDtypeStruct((N, 1), jnp.float32),
    ],
)


def _ln_update(h_ref, hws_ref, aggp_ref, dis_ref, cb_ref, g_ref, b_ref):
    dis = dis_ref[...]
    agg0 = aggp_ref[0, :, :] + aggp_ref[1, :, :]
    t = (agg0 + hws_ref[...]) * dis + cb_ref[...]
    xn = jnp.maximum(t, 0.0)
    hh = h_ref[...] + xn
    mu = jnp.mean(hh, axis=-1, keepdims=True)
    var = jnp.mean((hh - mu) ** 2, axis=-1, keepdims=True)
    return (hh - mu) * lax.rsqrt(var + 1e-5) * g_ref[...] + b_ref[...]


def _post_body(h_ref, hws_ref, aggp_ref, dis_ref, cb_ref, g_ref, b_ref, wn_ref,
               hn_ref, hwsn_ref):
    hn = _ln_update(h_ref, hws_ref, aggp_ref, dis_ref, cb_ref, g_ref, b_ref)
    hn_ref[...] = hn
    hw = jnp.dot(hn, wn_ref[...], preferred_element_type=jnp.float32)
    hwsn_ref[...] = hw * dis_ref[...]


_post = pl.pallas_call(
    _post_body,
    grid=(N // R,),
    in_specs=[
        pl.BlockSpec((R, D), lambda i: (i, 0)),
        pl.BlockSpec((R, D), lambda i: (i, 0)),
        pl.BlockSpec((NC, R, D), lambda i: (0, i, 0)),
        pl.BlockSpec((R, 1), lambda i: (i, 0)),
        pl.BlockSpec((1, D), lambda i: (0, 0)),
        pl.BlockSpec((1, D), lambda i: (0, 0)),
        pl.BlockSpec((1, D), lambda i: (0, 0)),
        pl.BlockSpec((D, D), lambda i: (0, 0)),
    ],
    out_specs=[
        pl.BlockSpec((R, D), lambda i: (i, 0)),
        pl.BlockSpec((R, D), lambda i: (i, 0)),
    ],
    out_shape=[
        jax.ShapeDtypeStruct((N, D), jnp.float32),
        jax.ShapeDtypeStruct((N, D), jnp.float32),
    ],
)


def _fin_body(h_ref, hws_ref, aggp_ref, dis_ref, cb_ref, g_ref, b_ref, ow_ref,
              batch_ref, out_ref, sums, cnts):
    i = pl.program_id(0)
    hn = _ln_update(h_ref, hws_ref, aggp_ref, dis_ref, cb_ref, g_ref, b_ref)

    @pl.when(i == 0)
    def _():
        sums[...] = jnp.zeros_like(sums)
        cnts[...] = jnp.zeros_like(cnts)

    bvec = batch_ref[0, 0, :]
    onehot = (lax.broadcasted_iota(jnp.int32, (G, R), 0)
              == bvec[None, :]).astype(jnp.float32)
    sums[...] += jnp.dot(onehot, hn, preferred_element_type=jnp.float32)
    cnts[...] += jnp.sum(onehot, axis=1, keepdims=True)

    @pl.when(i == pl.num_programs(0) - 1)
    def _():
        out_ref[...] = (jnp.dot(sums[...], ow_ref[...],
                                preferred_element_type=jnp.float32)
                        / jnp.maximum(cnts[...], 1.0))


_fin = pl.pallas_call(
    _fin_body,
    grid=(N // R,),
    in_specs=[
        pl.BlockSpec((R, D), lambda i: (i, 0)),
        pl.BlockSpec((R, D), lambda i: (i, 0)),
        pl.BlockSpec((NC, R, D), lambda i: (0, i, 0)),
        pl.BlockSpec((R, 1), lambda i: (i, 0)),
        pl.BlockSpec((1, D), lambda i: (0, 0)),
        pl.BlockSpec((1, D), lambda i: (0, 0)),
        pl.BlockSpec((1, D), lambda i: (0, 0)),
        pl.BlockSpec((D, 1), lambda i: (0, 0)),
        pl.BlockSpec((1, 1, R), lambda i: (i, 0, 0)),
    ],
    out_specs=pl.BlockSpec((G, 1), lambda i: (0, 0)),
    out_shape=jax.ShapeDtypeStruct((G, 1), jnp.float32),
    scratch_shapes=[
        pltpu.VMEM((G, D), jnp.float32),
        pltpu.VMEM((G, 1), jnp.float32),
    ],
)


def kernel(x, edge_index, batch, emb_W, emb_b, conv_W, conv_b, ln_g, ln_b, out_W):
    src = edge_index[0]
    dst = edge_index[1]
    zeros_nd = jnp.zeros((N, D), jnp.float32)
    zeros_16 = jnp.zeros((N, 16), jnp.float32)
    ones_16 = jnp.ones((CH, 16), jnp.float32)

    degp = _deg_kernel(dst, zeros_16, ones_16).reshape(NC, N, 16)
    h, hws, dis = _pro(x, emb_W, emb_b.reshape(1, D), conv_W[0], degp)

    out = None
    for i in range(3):
        aggp = _agg_kernel(hws, src, dst, zeros_nd).reshape(NC, N, D)
        cb = conv_b[i].reshape(1, D)
        g = ln_g[i].reshape(1, D)
        b = ln_b[i].reshape(1, D)
        if i < 2:
            h, hws = _post(h, hws, aggp, dis, cb, g, b, conv_W[i + 1])
        else:
            out = _fin(h, hws, aggp, dis, cb, g, b, out_W,
                       batch.reshape(N // R, 1, R))
    return out


# speedup vs baseline: 20.3632x; 1.4864x over previous
import functools

import jax
import jax.numpy as jnp
from jax import lax
from jax.experimental import pallas as pl
from jax.experimental.pallas import tpu as pltpu
from jax.experimental.pallas import tpu_sc as plsc

N = 10000
E = 320000
D = 128
G = 64

NC, NS = 2, 16
NW = NC * NS
CH = 128
EROWS = E // CH
KPW = EROWS // NW
XTRA = EROWS - NW * KPW
RS = 624
NT = N - NS * RS

_mesh = plsc.VectorSubcoreMesh(core_axis_name="c", subcore_axis_name="s")


@functools.partial(
    pl.kernel,
    out_type=jax.ShapeDtypeStruct((NC * N, 16), jnp.float32),
    mesh=_mesh,
    scratch_types=[
        pltpu.VMEM((2, CH), jnp.int32),
        pltpu.VMEM((CH, 16), jnp.float32),
        pltpu.VMEM_SHARED((N, 16), jnp.float32),
        pltpu.SemaphoreType.DMA,
    ],
    compiler_params=pltpu.CompilerParams(use_tc_tiling_on_sc=False),
)
def _deg_kernel(dst_hbm, zeros_hbm, ones_hbm, out_hbm, didx, ones_v, acc,
                ssem):
    c = lax.axis_index("c")
    s = lax.axis_index("s")
    w = c * NS + s
    k0 = w * KPW
    row0 = s * RS
    pltpu.sync_copy(zeros_hbm.at[pl.ds(row0, RS)], acc.at[pl.ds(row0, RS)])

    @pl.when(s == 0)
    def _():
        pltpu.sync_copy(zeros_hbm.at[pl.ds(NS * RS, NT)],
                        acc.at[pl.ds(NS * RS, NT)])

    pltpu.sync_copy(ones_hbm, ones_v)
    plsc.subcore_barrier()

    def s_start(b):
        pltpu.async_copy(ones_v, acc.at[didx.at[b]], ssem, add=True)

    def s_wait(b):
        pltpu.make_async_copy(ones_v, acc.at[didx.at[b]], ssem).wait()

    pltpu.sync_copy(dst_hbm.at[k0], didx.at[0])

    def body(t, carry):
        s_start(0)

        @pl.when(t > 0)
        def _():
            s_wait(1)
        pltpu.sync_copy(dst_hbm.at[k0 + 2 * t + 1], didx.at[1])
        s_start(1)
        s_wait(0)

        @pl.when(t < KPW // 2 - 1)
        def _():
            pltpu.sync_copy(dst_hbm.at[k0 + 2 * t + 2], didx.at[0])
        return carry

    lax.fori_loop(0, KPW // 2, body, 0)
    s_wait(1)

    @pl.when(w < XTRA)
    def _():
        pltpu.sync_copy(dst_hbm.at[NW * KPW + w], didx.at[0])
        pltpu.sync_copy(ones_v, acc.at[didx.at[0]], add=True)

    plsc.subcore_barrier()
    pltpu.sync_copy(acc.at[pl.ds(row0, RS)],
                    out_hbm.at[pl.ds(c * N + row0, RS)])

    @pl.when(s == 0)
    def _():
        pltpu.sync_copy(acc.at[pl.ds(NS * RS, NT)],
                        out_hbm.at[pl.ds(c * N + NS * RS, NT)])


@functools.partial(
    pl.kernel,
    out_type=jax.ShapeDtypeStruct((NC * N, D), jnp.float32),
    mesh=_mesh,
    scratch_types=[
        pltpu.VMEM((2, CH), jnp.int32),
        pltpu.VMEM((2, CH), jnp.int32),
        pltpu.VMEM((2, CH, D), jnp.float32),
        pltpu.VMEM_SHARED((N, D), jnp.float32),
        pltpu.SemaphoreType.DMA,
        pltpu.SemaphoreType.DMA,
    ],
)
def _agg_kernel(hws_hbm, src_hbm, dst_hbm, zeros_hbm, out_hbm,
                sidx, didx, rows, acc, gsem, ssem):
    c = lax.axis_index("c")
    s = lax.axis_index("s")
    w = c * NS + s
    k0 = w * KPW
    row0 = s * RS
    pltpu.sync_copy(zeros_hbm.at[pl.ds(row0, RS)], acc.at[pl.ds(row0, RS)])

    @pl.when(s == 0)
    def _():
        pltpu.sync_copy(zeros_hbm.at[pl.ds(NS * RS, NT)],
                        acc.at[pl.ds(NS * RS, NT)])

    plsc.subcore_barrier()

    def ld_idx(k, b):
        pltpu.sync_copy(src_hbm.at[k], sidx.at[b])
        pltpu.sync_copy(dst_hbm.at[k], didx.at[b])

    def g_start(b):
        pltpu.async_copy(hws_hbm.at[sidx.at[b]], rows.at[b], gsem)

    def g_wait(b):
        pltpu.make_async_copy(hws_hbm.at[sidx.at[b]], rows.at[b], gsem).wait()

    def s_start(b):
        pltpu.async_copy(rows.at[b], acc.at[didx.at[b]], ssem, add=True)

    def s_wait(b):
        pltpu.make_async_copy(rows.at[b], acc.at[didx.at[b]], ssem).wait()

    ld_idx(k0, 0)
    g_start(0)

    def body(t, carry):
        @pl.when(t > 0)
        def _():
            s_wait(1)
        ld_idx(k0 + 2 * t + 1, 1)
        g_start(1)
        g_wait(0)
        s_start(0)
        s_wait(0)

        @pl.when(t < KPW // 2 - 1)
        def _():
            ld_idx(k0 + 2 * t + 2, 0)
            g_start(0)
        g_wait(1)
        s_start(1)
        return carry

    lax.fori_loop(0, KPW // 2, body, 0)
    s_wait(1)

    @pl.when(w < XTRA)
    def _():
        ld_idx(NW * KPW + w, 0)
        g_start(0)
        g_wait(0)
        s_start(0)
        s_wait(0)

    plsc.subcore_barrier()
    pltpu.sync_copy(acc.at[pl.ds(row0, RS)],
                    out_hbm.at[pl.ds(c * N + row0, RS)])

    @pl.when(s == 0)
    def _():
        pltpu.sync_copy(acc.at[pl.ds(NS * RS, NT)],
                        out_hbm.at[pl.ds(c * N + NS * RS, NT)])


R = 1000


def _pro_body(x_ref, ew_ref, eb_ref, w0_ref, degp_ref, h_ref, hws_ref, dis_ref):
    deg = degp_ref[0, :, 0:1] + degp_ref[1, :, 0:1] + 1.0
    dis = lax.rsqrt(deg)
    h = jnp.dot(x_ref[...], ew_ref[...], preferred_element_type=jnp.float32)
    h = h + eb_ref[...]
    hw = jnp.dot(h, w0_ref[...], preferred_element_type=jnp.float32)
    h_ref[...] = h
    hws_ref[...] = hw * dis
    dis_ref[...] = dis


_pro = pl.pallas_call(
    _pro_body,
    grid=(N // R,),
    in_specs=[
        pl.BlockSpec((R, D), lambda i: (i, 0)),
        pl.BlockSpec((D, D), lambda i: (0, 0)),
        pl.BlockSpec((1, D), lambda i: (0, 0)),
        pl.BlockSpec((D, D), lambda i: (0, 0)),
        pl.BlockSpec((NC, R, 16), lambda i: (0, i, 0)),
    ],
    out_specs=[
        pl.BlockSpec((R, D), lambda i: (i, 0)),
        pl.BlockSpec((R, D), lambda i: (i, 0)),
        pl.BlockSpec((R, 1), lambda i: (i, 0)),
    ],
    out_shape=[
        jax.ShapeDtypeStruct((N, D), jnp.float32),
        jax.ShapeDtypeStruct((N, D), jnp.float32),
        jax.ShapeDtypeStruct((N, 1), jnp.float32),
    ],
)


def _ln_update(h_ref, hws_ref, aggp_ref, dis_ref, cb_ref, g_ref, b_ref):
    dis = dis_ref[...]
    agg0 = aggp_ref[0, :, :] + aggp_ref[1, :, :]
    t = (agg0 + hws_ref[...]) * dis + cb_ref[...]
    xn = jnp.maximum(t, 0.0)
    hh = h_ref[...] + xn
    mu = jnp.mean(hh, axis=-1, keepdims=True)
    var = jnp.mean((hh - mu) ** 2, axis=-1, keepdims=True)
    return (hh - mu) * lax.rsqrt(var + 1e-5) * g_ref[...] + b_ref[...]


def _post_body(h_ref, hws_ref, aggp_ref, dis_ref, cb_ref, g_ref, b_ref, wn_ref,
               hn_ref, hwsn_ref):
    hn = _ln_update(h_ref, hws_ref, aggp_ref, dis_ref, cb_ref, g_ref, b_ref)
    hn_ref[...] = hn
    hw = jnp.dot(hn, wn_ref[...], preferred_element_type=jnp.float32)
    hwsn_ref[...] = hw * dis_ref[...]


_post = pl.pallas_call(
    _post_body,
    grid=(N // R,),
    in_specs=[
        pl.BlockSpec((R, D), lambda i: (i, 0)),
        pl.BlockSpec((R, D), lambda i: (i, 0)),
        pl.BlockSpec((NC, R, D), lambda i: (0, i, 0)),
        pl.BlockSpec((R, 1), lambda i: (i, 0)),
        pl.BlockSpec((1, D), lambda i: (0, 0)),
        pl.BlockSpec((1, D), lambda i: (0, 0)),
        pl.BlockSpec((1, D), lambda i: (0, 0)),
        pl.BlockSpec((D, D), lambda i: (0, 0)),
    ],
    out_specs=[
        pl.BlockSpec((R, D), lambda i: (i, 0)),
        pl.BlockSpec((R, D), lambda i: (i, 0)),
    ],
    out_shape=[
        jax.ShapeDtypeStruct((N, D), jnp.float32),
        jax.ShapeDtypeStruct((N, D), jnp.float32),
    ],
)


def _fin_body(h_ref, hws_ref, aggp_ref, dis_ref, cb_ref, g_ref, b_ref, ow_ref,
              batch_ref, out_ref, sums, cnts):
    i = pl.program_id(0)
    hn = _ln_update(h_ref, hws_ref, aggp_ref, dis_ref, cb_ref, g_ref, b_ref)

    @pl.when(i == 0)
    def _():
        sums[...] = jnp.zeros_like(sums)
        cnts[...] = jnp.zeros_like(cnts)

    bvec = batch_ref[0, 0, :]
    onehot = (lax.broadcasted_iota(jnp.int32, (G, R), 0)
              == bvec[None, :]).astype(jnp.float32)
    sums[...] += jnp.dot(onehot, hn, preferred_element_type=jnp.float32)
    cnts[...] += jnp.sum(onehot, axis=1, keepdims=True)

    @pl.when(i == pl.num_programs(0) - 1)
    def _():
        out_ref[...] = (jnp.dot(sums[...], ow_ref[...],
                                preferred_element_type=jnp.float32)
                        / jnp.maximum(cnts[...], 1.0))


_fin = pl.pallas_call(
    _fin_body,
    grid=(N // R,),
    in_specs=[
        pl.BlockSpec((R, D), lambda i: (i, 0)),
        pl.BlockSpec((R, D), lambda i: (i, 0)),
        pl.BlockSpec((NC, R, D), lambda i: (0, i, 0)),
        pl.BlockSpec((R, 1), lambda i: (i, 0)),
        pl.BlockSpec((1, D), lambda i: (0, 0)),
        pl.BlockSpec((1, D), lambda i: (0, 0)),
        pl.BlockSpec((1, D), lambda i: (0, 0)),
        pl.BlockSpec((D, 1), lambda i: (0, 0)),
        pl.BlockSpec((1, 1, R), lambda i: (i, 0, 0)),
    ],
    out_specs=pl.BlockSpec((G, 1), lambda i: (0, 0)),
    out_shape=jax.ShapeDtypeStruct((G, 1), jnp.float32),
    scratch_shapes=[
        pltpu.VMEM((G, D), jnp.float32),
        pltpu.VMEM((G, 1), jnp.float32),
    ],
)


def kernel(x, edge_index, batch, emb_W, emb_b, conv_W, conv_b, ln_g, ln_b, out_W):
    src = edge_index[0].reshape(EROWS, CH)
    dst = edge_index[1].reshape(EROWS, CH)
    zeros_nd = jnp.zeros((N, D), jnp.float32)
    zeros_16 = jnp.zeros((N, 16), jnp.float32)
    ones_16 = jnp.ones((CH, 16), jnp.float32)

    degp = _deg_kernel(dst, zeros_16, ones_16).reshape(NC, N, 16)
    h, hws, dis = _pro(x, emb_W, emb_b.reshape(1, D), conv_W[0], degp)

    out = None
    for i in range(3):
        aggp = _agg_kernel(hws, src, dst, zeros_nd).reshape(NC, N, D)
        cb = conv_b[i].reshape(1, D)
        g = ln_g[i].reshape(1, D)
        b = ln_b[i].reshape(1, D)
        if i < 2:
            h, hws = _post(h, hws, aggp, dis, cb, g, b, conv_W[i + 1])
        else:
            out = _fin(h, hws, aggp, dis, cb, g, b, out_W,
                       batch.reshape(N // R, 1, R))
    return out


# speedup vs baseline: 23.8960x; 1.1735x over previous
import functools

import jax
import jax.numpy as jnp
from jax import lax
from jax.experimental import pallas as pl
from jax.experimental.pallas import tpu as pltpu
from jax.experimental.pallas import tpu_sc as plsc

N = 10000
E = 320000
D = 128
G = 64

NC, NS = 2, 16
NW = NC * NS
CH = 128
EROWS = E // CH
KPW = EROWS // NW
XTRA = EROWS - NW * KPW
RS = 624
NT = N - NS * RS

_mesh = plsc.VectorSubcoreMesh(core_axis_name="c", subcore_axis_name="s")


@functools.partial(
    pl.kernel,
    out_type=jax.ShapeDtypeStruct((NC * N, 16), jnp.float32),
    mesh=_mesh,
    scratch_types=[
        pltpu.VMEM((2, CH), jnp.int32),
        pltpu.VMEM((CH, 16), jnp.float32),
        pltpu.VMEM_SHARED((N, 16), jnp.float32),
        pltpu.SemaphoreType.DMA,
    ],
    compiler_params=pltpu.CompilerParams(use_tc_tiling_on_sc=False),
)
def _deg_kernel(dst_hbm, zeros_hbm, ones_hbm, out_hbm, didx, ones_v, acc,
                ssem):
    c = lax.axis_index("c")
    s = lax.axis_index("s")
    w = c * NS + s
    k0 = w * KPW
    row0 = s * RS
    pltpu.sync_copy(zeros_hbm.at[pl.ds(row0, RS)], acc.at[pl.ds(row0, RS)])

    @pl.when(s == 0)
    def _():
        pltpu.sync_copy(zeros_hbm.at[pl.ds(NS * RS, NT)],
                        acc.at[pl.ds(NS * RS, NT)])

    pltpu.sync_copy(ones_hbm, ones_v)
    plsc.subcore_barrier()

    def s_start(b):
        pltpu.async_copy(ones_v, acc.at[didx.at[b]], ssem, add=True)

    def s_wait(b):
        pltpu.make_async_copy(ones_v, acc.at[didx.at[b]], ssem).wait()

    pltpu.sync_copy(dst_hbm.at[k0], didx.at[0])

    def body(t, carry):
        s_start(0)

        @pl.when(t > 0)
        def _():
            s_wait(1)
        pltpu.sync_copy(dst_hbm.at[k0 + 2 * t + 1], didx.at[1])
        s_start(1)
        s_wait(0)

        @pl.when(t < KPW // 2 - 1)
        def _():
            pltpu.sync_copy(dst_hbm.at[k0 + 2 * t + 2], didx.at[0])
        return carry

    lax.fori_loop(0, KPW // 2, body, 0)
    s_wait(1)

    @pl.when(w < XTRA)
    def _():
        pltpu.sync_copy(dst_hbm.at[NW * KPW + w], didx.at[0])
        pltpu.sync_copy(ones_v, acc.at[didx.at[0]], add=True)

    plsc.subcore_barrier()
    pltpu.sync_copy(acc.at[pl.ds(row0, RS)],
                    out_hbm.at[pl.ds(c * N + row0, RS)])

    @pl.when(s == 0)
    def _():
        pltpu.sync_copy(acc.at[pl.ds(NS * RS, NT)],
                        out_hbm.at[pl.ds(c * N + NS * RS, NT)])


@functools.partial(
    pl.kernel,
    out_type=jax.ShapeDtypeStruct((NC * N, D), jnp.float32),
    mesh=_mesh,
    scratch_types=[
        pltpu.VMEM((3, CH), jnp.int32),
        pltpu.VMEM((3, CH), jnp.int32),
        pltpu.VMEM((3, CH, D), jnp.float32),
        pltpu.VMEM_SHARED((N, D), jnp.float32),
        pltpu.SemaphoreType.DMA,
        pltpu.SemaphoreType.DMA,
    ],
)
def _agg_kernel(hws_hbm, src_hbm, dst_hbm, zeros_hbm, out_hbm,
                sidx, didx, rows, acc, gsem, ssem):
    c = lax.axis_index("c")
    s = lax.axis_index("s")
    w = c * NS + s
    k0 = w * KPW
    row0 = s * RS
    pltpu.sync_copy(zeros_hbm.at[pl.ds(row0, RS)], acc.at[pl.ds(row0, RS)])

    @pl.when(s == 0)
    def _():
        pltpu.sync_copy(zeros_hbm.at[pl.ds(NS * RS, NT)],
                        acc.at[pl.ds(NS * RS, NT)])

    plsc.subcore_barrier()

    def ld_idx(k, b):
        pltpu.sync_copy(src_hbm.at[k], sidx.at[b])
        pltpu.sync_copy(dst_hbm.at[k], didx.at[b])

    def g_start(b):
        pltpu.async_copy(hws_hbm.at[sidx.at[b]], rows.at[b], gsem)

    def g_wait(b):
        pltpu.make_async_copy(hws_hbm.at[sidx.at[b]], rows.at[b], gsem).wait()

    def s_start(b):
        pltpu.async_copy(rows.at[b], acc.at[didx.at[b]], ssem, add=True)

    def s_wait(b):
        pltpu.make_async_copy(rows.at[b], acc.at[didx.at[b]], ssem).wait()

    ld_idx(k0, 0)
    g_start(0)
    ld_idx(k0 + 1, 1)
    g_start(1)
    TB = KPW // 3

    def body(t, carry):
        g_wait(0)
        s_start(0)

        @pl.when(t > 0)
        def _():
            s_wait(2)
        ld_idx(k0 + 3 * t + 2, 2)
        g_start(2)
        g_wait(1)
        s_start(1)
        s_wait(0)

        @pl.when(t < TB - 1)
        def _():
            ld_idx(k0 + 3 * t + 3, 0)
            g_start(0)
        g_wait(2)
        s_start(2)
        s_wait(1)

        @pl.when(t < TB - 1)
        def _():
            ld_idx(k0 + 3 * t + 4, 1)
            g_start(1)
        return carry

    lax.fori_loop(0, TB, body, 0)
    s_wait(2)

    @pl.when(w < XTRA)
    def _():
        ld_idx(NW * KPW + w, 0)
        g_start(0)
        g_wait(0)
        s_start(0)
        s_wait(0)

    plsc.subcore_barrier()
    pltpu.sync_copy(acc.at[pl.ds(row0, RS)],
                    out_hbm.at[pl.ds(c * N + row0, RS)])

    @pl.when(s == 0)
    def _():
        pltpu.sync_copy(acc.at[pl.ds(NS * RS, NT)],
                        out_hbm.at[pl.ds(c * N + NS * RS, NT)])


R = 1000


def _pro_body(x_ref, ew_ref, eb_ref, w0_ref, degp_ref, h_ref, hws_ref, dis_ref):
    deg = degp_ref[0, :, 0:1] + degp_ref[1, :, 0:1] + 1.0
    dis = lax.rsqrt(deg)
    h = jnp.dot(x_ref[...], ew_ref[...], preferred_element_type=jnp.float32)
    h = h + eb_ref[...]
    hw = jnp.dot(h, w0_ref[...], preferred_element_type=jnp.float32)
    h_ref[...] = h
    hws_ref[...] = hw * dis
    dis_ref[...] = dis


_pro = pl.pallas_call(
    _pro_body,
    grid=(N // R,),
    in_specs=[
        pl.BlockSpec((R, D), lambda i: (i, 0)),
        pl.BlockSpec((D, D), lambda i: (0, 0)),
        pl.BlockSpec((1, D), lambda i: (0, 0)),
        pl.BlockSpec((D, D), lambda i: (0, 0)),
        pl.BlockSpec((NC, R, 16), lambda i: (0, i, 0)),
    ],
    out_specs=[
        pl.BlockSpec((R, D), lambda i: (i, 0)),
        pl.BlockSpec((R, D), lambda i: (i, 0)),
        pl.BlockSpec((R, 1), lambda i: (i, 0)),
    ],
    out_shape=[
        jax.ShapeDtypeStruct((N, D), jnp.float32),
        jax.ShapeDtypeStruct((N, D), jnp.float32),
        jax.ShapeDtypeStruct((N, 1), jnp.float32),
    ],
)


def _ln_update(h_ref, hws_ref, aggp_ref, dis_ref, cb_ref, g_ref, b_ref):
    dis = dis_ref[...]
    agg0 = aggp_ref[0, :, :] + aggp_ref[1, :, :]
    t = (agg0 + hws_ref[...]) * dis + cb_ref[...]
    xn = jnp.maximum(t, 0.0)
    hh = h_ref[...] + xn
    mu = jnp.mean(hh, axis=-1, keepdims=True)
    var = jnp.mean((hh - mu) ** 2, axis=-1, keepdims=True)
    return (hh - mu) * lax.rsqrt(var + 1e-5) * g_ref[...] + b_ref[...]


def _post_body(h_ref, hws_ref, aggp_ref, dis_ref, cb_ref, g_ref, b_ref, wn_ref,
               hn_ref, hwsn_ref):
    hn = _ln_update(h_ref, hws_ref, aggp_ref, dis_ref, cb_ref, g_ref, b_ref)
    hn_ref[...] = hn
    hw = jnp.dot(hn, wn_ref[...], preferred_element_type=jnp.float32)
    hwsn_ref[...] = hw * dis_ref[...]


_post = pl.pallas_call(
    _post_body,
    grid=(N // R,),
    in_specs=[
        pl.BlockSpec((R, D), lambda i: (i, 0)),
        pl.BlockSpec((R, D), lambda i: (i, 0)),
        pl.BlockSpec((NC, R, D), lambda i: (0, i, 0)),
        pl.BlockSpec((R, 1), lambda i: (i, 0)),
        pl.BlockSpec((1, D), lambda i: (0, 0)),
        pl.BlockSpec((1, D), lambda i: (0, 0)),
        pl.BlockSpec((1, D), lambda i: (0, 0)),
        pl.BlockSpec((D, D), lambda i: (0, 0)),
    ],
    out_specs=[
        pl.BlockSpec((R, D), lambda i: (i, 0)),
        pl.BlockSpec((R, D), lambda i: (i, 0)),
    ],
    out_shape=[
        jax.ShapeDtypeStruct((N, D), jnp.float32),
        jax.ShapeDtypeStruct((N, D), jnp.float32),
    ],
)


def _fin_body(h_ref, hws_ref, aggp_ref, dis_ref, cb_ref, g_ref, b_ref, ow_ref,
              batch_ref, out_ref, sums, cnts):
    i = pl.program_id(0)
    hn = _ln_update(h_ref, hws_ref, aggp_ref, dis_ref, cb_ref, g_ref, b_ref)

    @pl.when(i == 0)
    def _():
        sums[...] = jnp.zeros_like(sums)
        cnts[...] = jnp.zeros_like(cnts)

    bvec = batch_ref[0, 0, :]
    onehot = (lax.broadcasted_iota(jnp.int32, (G, R), 0)
              == bvec[None, :]).astype(jnp.float32)
    sums[...] += jnp.dot(onehot, hn, preferred_element_type=jnp.float32)
    cnts[...] += jnp.sum(onehot, axis=1, keepdims=True)

    @pl.when(i == pl.num_programs(0) - 1)
    def _():
        out_ref[...] = (jnp.dot(sums[...], ow_ref[...],
                                preferred_element_type=jnp.float32)
                        / jnp.maximum(cnts[...], 1.0))


_fin = pl.pallas_call(
    _fin_body,
    grid=(N // R,),
    in_specs=[
        pl.BlockSpec((R, D), lambda i: (i, 0)),
        pl.BlockSpec((R, D), lambda i: (i, 0)),
        pl.BlockSpec((NC, R, D), lambda i: (0, i, 0)),
        pl.BlockSpec((R, 1), lambda i: (i, 0)),
        pl.BlockSpec((1, D), lambda i: (0, 0)),
        pl.BlockSpec((1, D), lambda i: (0, 0)),
        pl.BlockSpec((1, D), lambda i: (0, 0)),
        pl.BlockSpec((D, 1), lambda i: (0, 0)),
        pl.BlockSpec((1, 1, R), lambda i: (i, 0, 0)),
    ],
    out_specs=pl.BlockSpec((G, 1), lambda i: (0, 0)),
    out_shape=jax.ShapeDtypeStruct((G, 1), jnp.float32),
    scratch_shapes=[
        pltpu.VMEM((G, D), jnp.float32),
        pltpu.VMEM((G, 1), jnp.float32),
    ],
)


def kernel(x, edge_index, batch, emb_W, emb_b, conv_W, conv_b, ln_g, ln_b, out_W):
    src = edge_index[0].reshape(EROWS, CH)
    dst = edge_index[1].reshape(EROWS, CH)
    zeros_nd = jnp.zeros((N, D), jnp.float32)
    zeros_16 = jnp.zeros((N, 16), jnp.float32)
    ones_16 = jnp.ones((CH, 16), jnp.float32)

    degp = _deg_kernel(dst, zeros_16, ones_16).reshape(NC, N, 16)
    h, hws, dis = _pro(x, emb_W, emb_b.reshape(1, D), conv_W[0], degp)

    out = None
    for i in range(3):
        aggp = _agg_kernel(hws, src, dst, zeros_nd).reshape(NC, N, D)
        cb = conv_b[i].reshape(1, D)
        g = ln_g[i].reshape(1, D)
        b = ln_b[i].reshape(1, D)
        if i < 2:
            h, hws = _post(h, hws, aggp, dis, cb, g, b, conv_W[i + 1])
        else:
            out = _fin(h, hws, aggp, dis, cb, g, b, out_W,
                       batch.reshape(N // R, 1, R))
    return out


# speedup vs baseline: 25.3624x; 1.0614x over previous
import functools

import jax
import jax.numpy as jnp
from jax import lax
from jax.experimental import pallas as pl
from jax.experimental.pallas import tpu as pltpu
from jax.experimental.pallas import tpu_sc as plsc

N = 10000
E = 320000
D = 128
G = 64

NC, NS = 2, 16
NW = NC * NS
CH = 64
EROWS = E // CH
KPW = EROWS // NW
XTRA = EROWS - NW * KPW
RS = 624
NT = N - NS * RS

_mesh = plsc.VectorSubcoreMesh(core_axis_name="c", subcore_axis_name="s")


@functools.partial(
    pl.kernel,
    out_type=jax.ShapeDtypeStruct((NC * N, 16), jnp.float32),
    mesh=_mesh,
    scratch_types=[
        pltpu.VMEM((2, CH), jnp.int32),
        pltpu.VMEM((CH, 16), jnp.float32),
        pltpu.VMEM_SHARED((N, 16), jnp.float32),
        pltpu.SemaphoreType.DMA,
    ],
    compiler_params=pltpu.CompilerParams(use_tc_tiling_on_sc=False),
)
def _deg_kernel(dst_hbm, zeros_hbm, ones_hbm, out_hbm, didx, ones_v, acc,
                ssem):
    c = lax.axis_index("c")
    s = lax.axis_index("s")
    w = c * NS + s
    k0 = w * KPW
    row0 = s * RS
    pltpu.sync_copy(zeros_hbm.at[pl.ds(row0, RS)], acc.at[pl.ds(row0, RS)])

    @pl.when(s == 0)
    def _():
        pltpu.sync_copy(zeros_hbm.at[pl.ds(NS * RS, NT)],
                        acc.at[pl.ds(NS * RS, NT)])

    pltpu.sync_copy(ones_hbm, ones_v)
    plsc.subcore_barrier()

    def s_start(b):
        pltpu.async_copy(ones_v, acc.at[didx.at[b]], ssem, add=True)

    def s_wait(b):
        pltpu.make_async_copy(ones_v, acc.at[didx.at[b]], ssem).wait()

    pltpu.sync_copy(dst_hbm.at[k0], didx.at[0])

    def body(t, carry):
        s_start(0)

        @pl.when(t > 0)
        def _():
            s_wait(1)
        pltpu.sync_copy(dst_hbm.at[k0 + 2 * t + 1], didx.at[1])
        s_start(1)
        s_wait(0)

        @pl.when(t < KPW // 2 - 1)
        def _():
            pltpu.sync_copy(dst_hbm.at[k0 + 2 * t + 2], didx.at[0])
        return carry

    lax.fori_loop(0, KPW // 2, body, 0)
    s_wait(1)

    @pl.when(w < XTRA)
    def _():
        pltpu.sync_copy(dst_hbm.at[NW * KPW + w], didx.at[0])
        pltpu.sync_copy(ones_v, acc.at[didx.at[0]], add=True)

    plsc.subcore_barrier()
    pltpu.sync_copy(acc.at[pl.ds(row0, RS)],
                    out_hbm.at[pl.ds(c * N + row0, RS)])

    @pl.when(s == 0)
    def _():
        pltpu.sync_copy(acc.at[pl.ds(NS * RS, NT)],
                        out_hbm.at[pl.ds(c * N + NS * RS, NT)])


@functools.partial(
    pl.kernel,
    out_type=jax.ShapeDtypeStruct((NC * N, D), jnp.float32),
    mesh=_mesh,
    scratch_types=[
        pltpu.VMEM((6, CH), jnp.int32),
        pltpu.VMEM((6, CH), jnp.int32),
        pltpu.VMEM((3, CH, D), jnp.float32),
        pltpu.VMEM_SHARED((N, D), jnp.float32),
        pltpu.SemaphoreType.DMA,
        pltpu.SemaphoreType.DMA,
        pltpu.SemaphoreType.DMA,
    ],
)
def _agg_kernel(hws_hbm, src_hbm, dst_hbm, zeros_hbm, out_hbm,
                sidx, didx, rows, acc, gsem, ssem, isem):
    c = lax.axis_index("c")
    s = lax.axis_index("s")
    w = c * NS + s
    k0 = w * KPW
    row0 = s * RS
    pltpu.sync_copy(zeros_hbm.at[pl.ds(row0, RS)], acc.at[pl.ds(row0, RS)])

    @pl.when(s == 0)
    def _():
        pltpu.sync_copy(zeros_hbm.at[pl.ds(NS * RS, NT)],
                        acc.at[pl.ds(NS * RS, NT)])

    plsc.subcore_barrier()
    klast = jnp.where(w < XTRA, KPW, KPW - 1)

    def erow(k):
        return jnp.where(k < KPW, k0 + k, NW * KPW + w)

    def i_start(k, j):
        pltpu.async_copy(src_hbm.at[erow(k)], sidx.at[j], isem)
        pltpu.async_copy(dst_hbm.at[erow(k)], didx.at[j], isem)

    def i_wait(k, j):
        pltpu.make_async_copy(src_hbm.at[erow(k)], sidx.at[j], isem).wait()
        pltpu.make_async_copy(dst_hbm.at[erow(k)], didx.at[j], isem).wait()

    def g_start(j, b):
        pltpu.async_copy(hws_hbm.at[sidx.at[j]], rows.at[b], gsem)

    def g_wait(j, b):
        pltpu.make_async_copy(hws_hbm.at[sidx.at[j]], rows.at[b], gsem).wait()

    def s_start(j, b):
        pltpu.async_copy(rows.at[b], acc.at[didx.at[j]], ssem, add=True)

    def s_wait(j, b):
        pltpu.make_async_copy(rows.at[b], acc.at[didx.at[j]], ssem).wait()

    pltpu.sync_copy(src_hbm.at[erow(0)], sidx.at[0])
    pltpu.sync_copy(dst_hbm.at[erow(0)], didx.at[0])
    pltpu.sync_copy(src_hbm.at[erow(1)], sidx.at[1])
    pltpu.sync_copy(dst_hbm.at[erow(1)], didx.at[1])
    i_start(2, 2)
    i_start(3, 3)
    g_start(0, 0)
    g_start(1, 1)

    def step(t, u):
        k = 6 * t + u
        b = u % 3
        if u == 0:
            @pl.when(t > 0)
            def _():
                s_wait((u - 1) % 6, (u - 1) % 3)
        else:
            s_wait((u - 1) % 6, (u - 1) % 3)

        @pl.when(k + 2 <= klast)
        def _():
            i_wait(k + 2, (u + 2) % 6)
            g_start((u + 2) % 6, (u + 2) % 3)

        @pl.when(k + 4 <= klast)
        def _():
            i_start(k + 4, (u + 4) % 6)
        g_wait(u, b)
        s_start(u, b)

    def body(t, carry):
        for u in range(6):
            step(t, u)
        return carry

    lax.fori_loop(0, KPW // 6, body, 0)

    @pl.when(w < XTRA)
    def _():
        g_wait(0, 0)
        s_start(0, 0)
        s_wait(0, 0)

    s_wait((KPW - 1) % 6, (KPW - 1) % 3)

    plsc.subcore_barrier()
    pltpu.sync_copy(acc.at[pl.ds(row0, RS)],
                    out_hbm.at[pl.ds(c * N + row0, RS)])

    @pl.when(s == 0)
    def _():
        pltpu.sync_copy(acc.at[pl.ds(NS * RS, NT)],
                        out_hbm.at[pl.ds(c * N + NS * RS, NT)])


R = 1000


def _pro_body(x_ref, ew_ref, eb_ref, w0_ref, degp_ref, h_ref, hws_ref, dis_ref):
    deg = degp_ref[0, :, 0:1] + degp_ref[1, :, 0:1] + 1.0
    dis = lax.rsqrt(deg)
    h = jnp.dot(x_ref[...], ew_ref[...], preferred_element_type=jnp.float32)
    h = h + eb_ref[...]
    hw = jnp.dot(h, w0_ref[...], preferred_element_type=jnp.float32)
    h_ref[...] = h
    hws_ref[...] = hw * dis
    dis_ref[...] = dis


_pro = pl.pallas_call(
    _pro_body,
    grid=(N // R,),
    in_specs=[
        pl.BlockSpec((R, D), lambda i: (i, 0)),
        pl.BlockSpec((D, D), lambda i: (0, 0)),
        pl.BlockSpec((1, D), lambda i: (0, 0)),
        pl.BlockSpec((D, D), lambda i: (0, 0)),
        pl.BlockSpec((NC, R, 16), lambda i: (0, i, 0)),
    ],
    out_specs=[
        pl.BlockSpec((R, D), lambda i: (i, 0)),
        pl.BlockSpec((R, D), lambda i: (i, 0)),
        pl.BlockSpec((R, 1), lambda i: (i, 0)),
    ],
    out_shape=[
        jax.ShapeDtypeStruct((N, D), jnp.float32),
        jax.ShapeDtypeStruct((N, D), jnp.float32),
        jax.ShapeDtypeStruct((N, 1), jnp.float32),
    ],
)


def _ln_update(h_ref, hws_ref, aggp_ref, dis_ref, cb_ref, g_ref, b_ref):
    dis = dis_ref[...]
    agg0 = aggp_ref[0, :, :] + aggp_ref[1, :, :]
    t = (agg0 + hws_ref[...]) * dis + cb_ref[...]
    xn = jnp.maximum(t, 0.0)
    hh = h_ref[...] + xn
    mu = jnp.mean(hh, axis=-1, keepdims=True)
    var = jnp.mean((hh - mu) ** 2, axis=-1, keepdims=True)
    return (hh - mu) * lax.rsqrt(var + 1e-5) * g_ref[...] + b_ref[...]


def _post_body(h_ref, hws_ref, aggp_ref, dis_ref, cb_ref, g_ref, b_ref, wn_ref,
               hn_ref, hwsn_ref):
    hn = _ln_update(h_ref, hws_ref, aggp_ref, dis_ref, cb_ref, g_ref, b_ref)
    hn_ref[...] = hn
    hw = jnp.dot(hn, wn_ref[...], preferred_element_type=jnp.float32)
    hwsn_ref[...] = hw * dis_ref[...]


_post = pl.pallas_call(
    _post_body,
    grid=(N // R,),
    in_specs=[
        pl.BlockSpec((R, D), lambda i: (i, 0)),
        pl.BlockSpec((R, D), lambda i: (i, 0)),
        pl.BlockSpec((NC, R, D), lambda i: (0, i, 0)),
        pl.BlockSpec((R, 1), lambda i: (i, 0)),
        pl.BlockSpec((1, D), lambda i: (0, 0)),
        pl.BlockSpec((1, D), lambda i: (0, 0)),
        pl.BlockSpec((1, D), lambda i: (0, 0)),
        pl.BlockSpec((D, D), lambda i: (0, 0)),
    ],
    out_specs=[
        pl.BlockSpec((R, D), lambda i: (i, 0)),
        pl.BlockSpec((R, D), lambda i: (i, 0)),
    ],
    out_shape=[
        jax.ShapeDtypeStruct((N, D), jnp.float32),
        jax.ShapeDtypeStruct((N, D), jnp.float32),
    ],
)


def _fin_body(h_ref, hws_ref, aggp_ref, dis_ref, cb_ref, g_ref, b_ref, ow_ref,
              batch_ref, out_ref, sums, cnts):
    i = pl.program_id(0)
    hn = _ln_update(h_ref, hws_ref, aggp_ref, dis_ref, cb_ref, g_ref, b_ref)

    @pl.when(i == 0)
    def _():
        sums[...] = jnp.zeros_like(sums)
        cnts[...] = jnp.zeros_like(cnts)

    bvec = batch_ref[0, 0, :]
    onehot = (lax.broadcasted_iota(jnp.int32, (G, R), 0)
              == bvec[None, :]).astype(jnp.float32)
    sums[...] += jnp.dot(onehot, hn, preferred_element_type=jnp.float32)
    cnts[...] += jnp.sum(onehot, axis=1, keepdims=True)

    @pl.when(i == pl.num_programs(0) - 1)
    def _():
        out_ref[...] = (jnp.dot(sums[...], ow_ref[...],
                                preferred_element_type=jnp.float32)
                        / jnp.maximum(cnts[...], 1.0))


_fin = pl.pallas_call(
    _fin_body,
    grid=(N // R,),
    in_specs=[
        pl.BlockSpec((R, D), lambda i: (i, 0)),
        pl.BlockSpec((R, D), lambda i: (i, 0)),
        pl.BlockSpec((NC, R, D), lambda i: (0, i, 0)),
        pl.BlockSpec((R, 1), lambda i: (i, 0)),
        pl.BlockSpec((1, D), lambda i: (0, 0)),
        pl.BlockSpec((1, D), lambda i: (0, 0)),
        pl.BlockSpec((1, D), lambda i: (0, 0)),
        pl.BlockSpec((D, 1), lambda i: (0, 0)),
        pl.BlockSpec((1, 1, R), lambda i: (i, 0, 0)),
    ],
    out_specs=pl.BlockSpec((G, 1), lambda i: (0, 0)),
    out_shape=jax.ShapeDtypeStruct((G, 1), jnp.float32),
    scratch_shapes=[
        pltpu.VMEM((G, D), jnp.float32),
        pltpu.VMEM((G, 1), jnp.float32),
    ],
)


def kernel(x, edge_index, batch, emb_W, emb_b, conv_W, conv_b, ln_g, ln_b, out_W):
    src = edge_index[0].reshape(EROWS, CH)
    dst = edge_index[1].reshape(EROWS, CH)
    zeros_nd = jnp.zeros((N, D), jnp.float32)
    zeros_16 = jnp.zeros((N, 16), jnp.float32)
    ones_16 = jnp.ones((CH, 16), jnp.float32)

    degp = _deg_kernel(dst, zeros_16, ones_16).reshape(NC, N, 16)
    h, hws, dis = _pro(x, emb_W, emb_b.reshape(1, D), conv_W[0], degp)

    out = None
    for i in range(3):
        aggp = _agg_kernel(hws, src, dst, zeros_nd).reshape(NC, N, D)
        cb = conv_b[i].reshape(1, D)
        g = ln_g[i].reshape(1, D)
        b = ln_b[i].reshape(1, D)
        if i < 2:
            h, hws = _post(h, hws, aggp, dis, cb, g, b, conv_W[i + 1])
        else:
            out = _fin(h, hws, aggp, dis, cb, g, b, out_W,
                       batch.reshape(N // R, 1, R))
    return out


# speedup vs baseline: 26.5655x; 1.0474x over previous
import functools

import jax
import jax.numpy as jnp
from jax import lax
from jax.experimental import pallas as pl
from jax.experimental.pallas import tpu as pltpu
from jax.experimental.pallas import tpu_sc as plsc

N = 10000
E = 320000
D = 128
G = 64

NC, NS = 2, 16
NW = NC * NS
CH = 64
EROWS = E // CH
KPW = EROWS // NW
XTRA = EROWS - NW * KPW
RS = 624
NT = N - NS * RS

_mesh = plsc.VectorSubcoreMesh(core_axis_name="c", subcore_axis_name="s")


@functools.partial(
    pl.kernel,
    out_type=jax.ShapeDtypeStruct((NC * N, 16), jnp.float32),
    mesh=_mesh,
    scratch_types=[
        pltpu.VMEM((2, CH), jnp.int32),
        pltpu.VMEM((CH, 16), jnp.float32),
        pltpu.VMEM_SHARED((N, 16), jnp.float32),
        pltpu.SemaphoreType.DMA,
    ],
    compiler_params=pltpu.CompilerParams(use_tc_tiling_on_sc=False),
)
def _deg_kernel(dst_hbm, zeros_hbm, ones_hbm, out_hbm, didx, ones_v, acc,
                ssem):
    c = lax.axis_index("c")
    s = lax.axis_index("s")
    w = c * NS + s
    k0 = w * KPW
    row0 = s * RS
    pltpu.sync_copy(zeros_hbm.at[pl.ds(row0, RS)], acc.at[pl.ds(row0, RS)])

    @pl.when(s == 0)
    def _():
        pltpu.sync_copy(zeros_hbm.at[pl.ds(NS * RS, NT)],
                        acc.at[pl.ds(NS * RS, NT)])

    pltpu.sync_copy(ones_hbm, ones_v)
    plsc.subcore_barrier()

    def s_start(b):
        pltpu.async_copy(ones_v, acc.at[didx.at[b]], ssem, add=True)

    def s_wait(b):
        pltpu.make_async_copy(ones_v, acc.at[didx.at[b]], ssem).wait()

    pltpu.sync_copy(dst_hbm.at[k0], didx.at[0])

    def body(t, carry):
        s_start(0)

        @pl.when(t > 0)
        def _():
            s_wait(1)
        pltpu.sync_copy(dst_hbm.at[k0 + 2 * t + 1], didx.at[1])
        s_start(1)
        s_wait(0)

        @pl.when(t < KPW // 2 - 1)
        def _():
            pltpu.sync_copy(dst_hbm.at[k0 + 2 * t + 2], didx.at[0])
        return carry

    lax.fori_loop(0, KPW // 2, body, 0)
    s_wait(1)

    @pl.when(w < XTRA)
    def _():
        pltpu.sync_copy(dst_hbm.at[NW * KPW + w], didx.at[0])
        pltpu.sync_copy(ones_v, acc.at[didx.at[0]], add=True)

    plsc.subcore_barrier()
    pltpu.sync_copy(acc.at[pl.ds(row0, RS)],
                    out_hbm.at[pl.ds(c * N + row0, RS)])

    @pl.when(s == 0)
    def _():
        pltpu.sync_copy(acc.at[pl.ds(NS * RS, NT)],
                        out_hbm.at[pl.ds(c * N + NS * RS, NT)])


@functools.partial(
    pl.kernel,
    out_type=jax.ShapeDtypeStruct((NC * N, D), jnp.float32),
    mesh=_mesh,
    scratch_types=[
        pltpu.VMEM((6, CH), jnp.int32),
        pltpu.VMEM((3, CH), jnp.int32),
        pltpu.VMEM((6, CH, D), jnp.float32),
        pltpu.VMEM_SHARED((N, D), jnp.float32),
        pltpu.SemaphoreType.DMA,
        pltpu.SemaphoreType.DMA,
        pltpu.SemaphoreType.DMA,
        pltpu.SemaphoreType.DMA,
    ],
)
def _agg_kernel(hws_hbm, src_hbm, dst_hbm, zeros_hbm, out_hbm,
                sidx, didx, rows, acc, gsem, ssem, issem, idsem):
    c = lax.axis_index("c")
    s = lax.axis_index("s")
    w = c * NS + s
    k0 = w * KPW
    row0 = s * RS
    pltpu.sync_copy(zeros_hbm.at[pl.ds(row0, RS)], acc.at[pl.ds(row0, RS)])

    @pl.when(s == 0)
    def _():
        pltpu.sync_copy(zeros_hbm.at[pl.ds(NS * RS, NT)],
                        acc.at[pl.ds(NS * RS, NT)])

    plsc.subcore_barrier()
    klast = jnp.where(w < XTRA, KPW, KPW - 1)

    def erow(k):
        return jnp.where(k < KPW, k0 + k, NW * KPW + w)

    def is_start(k, j):
        pltpu.async_copy(src_hbm.at[erow(k)], sidx.at[j], issem)

    def is_wait(k, j):
        pltpu.make_async_copy(src_hbm.at[erow(k)], sidx.at[j], issem).wait()

    def id_start(k, j):
        pltpu.async_copy(dst_hbm.at[erow(k)], didx.at[j], idsem)

    def id_wait(k, j):
        pltpu.make_async_copy(dst_hbm.at[erow(k)], didx.at[j], idsem).wait()

    def g_start(j, b):
        pltpu.async_copy(hws_hbm.at[sidx.at[j]], rows.at[b], gsem)

    def g_wait(j, b):
        pltpu.make_async_copy(hws_hbm.at[sidx.at[j]], rows.at[b], gsem).wait()

    def s_start(j, b):
        pltpu.async_copy(rows.at[b], acc.at[didx.at[j]], ssem, add=True)

    def s_wait(j, b):
        pltpu.make_async_copy(rows.at[b], acc.at[didx.at[j]], ssem).wait()

    pltpu.sync_copy(src_hbm.at[erow(0)], sidx.at[0])
    pltpu.sync_copy(src_hbm.at[erow(1)], sidx.at[1])
    pltpu.sync_copy(src_hbm.at[erow(2)], sidx.at[2])
    is_start(3, 3)
    is_start(4, 4)
    id_start(0, 0)
    id_start(1, 1)
    g_start(0, 0)
    g_start(1, 1)
    g_start(2, 2)

    def step(t, u):
        k = 12 * t + u
        b = u % 6
        j3 = u % 3
        if u == 0:
            @pl.when(t > 0)
            def _():
                s_wait(2, 5)
        else:
            s_wait((u - 1) % 3, (u - 1) % 6)

        @pl.when(k + 3 <= klast)
        def _():
            is_wait(k + 3, (u + 3) % 6)
            g_start((u + 3) % 6, (u + 3) % 6)

        @pl.when(k + 5 <= klast)
        def _():
            is_start(k + 5, (u + 5) % 6)

        @pl.when(k + 2 <= klast)
        def _():
            id_start(k + 2, (u + 2) % 3)
        g_wait(b, b)
        id_wait(k, j3)
        s_start(j3, b)

    def body(t, carry):
        for u in range(12):
            step(t, u)
        return carry

    lax.fori_loop(0, KPW // 12, body, 0)

    @pl.when(w < XTRA)
    def _():
        g_wait(0, 0)
        id_wait(KPW, 0)
        s_start(0, 0)
        s_wait(0, 0)

    s_wait((KPW - 1) % 3, (KPW - 1) % 6)

    plsc.subcore_barrier()
    pltpu.sync_copy(acc.at[pl.ds(row0, RS)],
                    out_hbm.at[pl.ds(c * N + row0, RS)])

    @pl.when(s == 0)
    def _():
        pltpu.sync_copy(acc.at[pl.ds(NS * RS, NT)],
                        out_hbm.at[pl.ds(c * N + NS * RS, NT)])


R = 1000


def _pro_body(x_ref, ew_ref, eb_ref, w0_ref, degp_ref, h_ref, hws_ref, dis_ref):
    deg = degp_ref[0, :, 0:1] + degp_ref[1, :, 0:1] + 1.0
    dis = lax.rsqrt(deg)
    h = jnp.dot(x_ref[...], ew_ref[...], preferred_element_type=jnp.float32)
    h = h + eb_ref[...]
    hw = jnp.dot(h, w0_ref[...], preferred_element_type=jnp.float32)
    h_ref[...] = h
    hws_ref[...] = hw * dis
    dis_ref[...] = dis


_pro = pl.pallas_call(
    _pro_body,
    grid=(N // R,),
    in_specs=[
        pl.BlockSpec((R, D), lambda i: (i, 0)),
        pl.BlockSpec((D, D), lambda i: (0, 0)),
        pl.BlockSpec((1, D), lambda i: (0, 0)),
        pl.BlockSpec((D, D), lambda i: (0, 0)),
        pl.BlockSpec((NC, R, 16), lambda i: (0, i, 0)),
    ],
    out_specs=[
        pl.BlockSpec((R, D), lambda i: (i, 0)),
        pl.BlockSpec((R, D), lambda i: (i, 0)),
        pl.BlockSpec((R, 1), lambda i: (i, 0)),
    ],
    out_shape=[
        jax.ShapeDtypeStruct((N, D), jnp.float32),
        jax.ShapeDtypeStruct((N, D), jnp.float32),
        jax.ShapeDtypeStruct((N, 1), jnp.float32),
    ],
)


def _ln_update(h_ref, hws_ref, aggp_ref, dis_ref, cb_ref, g_ref, b_ref):
    dis = dis_ref[...]
    agg0 = aggp_ref[0, :, :] + aggp_ref[1, :, :]
    t = (agg0 + hws_ref[...]) * dis + cb_ref[...]
    xn = jnp.maximum(t, 0.0)
    hh = h_ref[...] + xn
    mu = jnp.mean(hh, axis=-1, keepdims=True)
    var = jnp.mean((hh - mu) ** 2, axis=-1, keepdims=True)
    return (hh - mu) * lax.rsqrt(var + 1e-5) * g_ref[...] + b_ref[...]


def _post_body(h_ref, hws_ref, aggp_ref, dis_ref, cb_ref, g_ref, b_ref, wn_ref,
               hn_ref, hwsn_ref):
    hn = _ln_update(h_ref, hws_ref, aggp_ref, dis_ref, cb_ref, g_ref, b_ref)
    hn_ref[...] = hn
    hw = jnp.dot(hn, wn_ref[...], preferred_element_type=jnp.float32)
    hwsn_ref[...] = hw * dis_ref[...]


_post = pl.pallas_call(
    _post_body,
    grid=(N // R,),
    in_specs=[
        pl.BlockSpec((R, D), lambda i: (i, 0)),
        pl.BlockSpec((R, D), lambda i: (i, 0)),
        pl.BlockSpec((NC, R, D), lambda i: (0, i, 0)),
        pl.BlockSpec((R, 1), lambda i: (i, 0)),
        pl.BlockSpec((1, D), lambda i: (0, 0)),
        pl.BlockSpec((1, D), lambda i: (0, 0)),
        pl.BlockSpec((1, D), lambda i: (0, 0)),
        pl.BlockSpec((D, D), lambda i: (0, 0)),
    ],
    out_specs=[
        pl.BlockSpec((R, D), lambda i: (i, 0)),
        pl.BlockSpec((R, D), lambda i: (i, 0)),
    ],
    out_shape=[
        jax.ShapeDtypeStruct((N, D), jnp.float32),
        jax.ShapeDtypeStruct((N, D), jnp.float32),
    ],
)


def _fin_body(h_ref, hws_ref, aggp_ref, dis_ref, cb_ref, g_ref, b_ref, ow_ref,
              batch_ref, out_ref, sums, cnts):
    i = pl.program_id(0)
    hn = _ln_update(h_ref, hws_ref, aggp_ref, dis_ref, cb_ref, g_ref, b_ref)

    @pl.when(i == 0)
    def _():
        sums[...] = jnp.zeros_like(sums)
        cnts[...] = jnp.zeros_like(cnts)

    bvec = batch_ref[0, 0, :]
    onehot = (lax.broadcasted_iota(jnp.int32, (G, R), 0)
              == bvec[None, :]).astype(jnp.float32)
    sums[...] += jnp.dot(onehot, hn, preferred_element_type=jnp.float32)
    cnts[...] += jnp.sum(onehot, axis=1, keepdims=True)

    @pl.when(i == pl.num_programs(0) - 1)
    def _():
        out_ref[...] = (jnp.dot(sums[...], ow_ref[...],
                                preferred_element_type=jnp.float32)
                        / jnp.maximum(cnts[...], 1.0))


_fin = pl.pallas_call(
    _fin_body,
    grid=(N // R,),
    in_specs=[
        pl.BlockSpec((R, D), lambda i: (i, 0)),
        pl.BlockSpec((R, D), lambda i: (i, 0)),
        pl.BlockSpec((NC, R, D), lambda i: (0, i, 0)),
        pl.BlockSpec((R, 1), lambda i: (i, 0)),
        pl.BlockSpec((1, D), lambda i: (0, 0)),
        pl.BlockSpec((1, D), lambda i: (0, 0)),
        pl.BlockSpec((1, D), lambda i: (0, 0)),
        pl.BlockSpec((D, 1), lambda i: (0, 0)),
        pl.BlockSpec((1, 1, R), lambda i: (i, 0, 0)),
    ],
    out_specs=pl.BlockSpec((G, 1), lambda i: (0, 0)),
    out_shape=jax.ShapeDtypeStruct((G, 1), jnp.float32),
    scratch_shapes=[
        pltpu.VMEM((G, D), jnp.float32),
        pltpu.VMEM((G, 1), jnp.float32),
    ],
)


def kernel(x, edge_index, batch, emb_W, emb_b, conv_W, conv_b, ln_g, ln_b, out_W):
    src = edge_index[0].reshape(EROWS, CH)
    dst = edge_index[1].reshape(EROWS, CH)
    zeros_nd = jnp.zeros((N, D), jnp.float32)
    zeros_16 = jnp.zeros((N, 16), jnp.float32)
    ones_16 = jnp.ones((CH, 16), jnp.float32)

    degp = _deg_kernel(dst, zeros_16, ones_16).reshape(NC, N, 16)
    h, hws, dis = _pro(x, emb_W, emb_b.reshape(1, D), conv_W[0], degp)

    out = None
    for i in range(3):
        aggp = _agg_kernel(hws, src, dst, zeros_nd).reshape(NC, N, D)
        cb = conv_b[i].reshape(1, D)
        g = ln_g[i].reshape(1, D)
        b = ln_b[i].reshape(1, D)
        if i < 2:
            h, hws = _post(h, hws, aggp, dis, cb, g, b, conv_W[i + 1])
        else:
            out = _fin(h, hws, aggp, dis, cb, g, b, out_W,
                       batch.reshape(N // R, 1, R))
    return out


# speedup vs baseline: 30.4695x; 1.1470x over previous
import functools

import jax
import jax.numpy as jnp
from jax import lax
from jax.experimental import pallas as pl
from jax.experimental.pallas import tpu as pltpu
from jax.experimental.pallas import tpu_sc as plsc

N = 10000
E = 320000
D = 128
G = 64

NC, NS = 2, 16
NW = NC * NS
CH = 64
EROWS = E // CH
KPW = EROWS // NW
XTRA = EROWS - NW * KPW
RS = 624
NT = N - NS * RS

DCH = 128
DROWS = E // DCH
DKPW = DROWS // NW
DXTRA = DROWS - NW * DKPW
DDEPTH = 6

_mesh = plsc.VectorSubcoreMesh(core_axis_name="c", subcore_axis_name="s")


@functools.partial(
    pl.kernel,
    out_type=jax.ShapeDtypeStruct((NC * N, 16), jnp.float32),
    mesh=_mesh,
    scratch_types=[
        pltpu.VMEM((DKPW + 1, DCH), jnp.int32),
        pltpu.VMEM((DCH, 16), jnp.float32),
        pltpu.VMEM_SHARED((N, 16), jnp.float32),
        pltpu.SemaphoreType.DMA,
        pltpu.SemaphoreType.DMA,
    ],
    compiler_params=pltpu.CompilerParams(use_tc_tiling_on_sc=False),
)
def _deg_kernel(dst_hbm, zeros_hbm, ones_hbm, out_hbm, didx, ones_v, acc,
                isem, ssem):
    c = lax.axis_index("c")
    s = lax.axis_index("s")
    w = c * NS + s
    k0 = w * DKPW
    row0 = s * RS
    kcnt = jnp.where(w < DXTRA, DKPW + 1, DKPW)

    def drow(k):
        return jnp.where(k < DKPW, k0 + k, NW * DKPW + w)

    def i_start(k, carry):
        pltpu.async_copy(dst_hbm.at[drow(k)], didx.at[k], isem)
        return carry

    def i_drain(k, carry):
        pltpu.make_async_copy(dst_hbm.at[drow(k)], didx.at[k], isem).wait()
        return carry

    lax.fori_loop(0, kcnt, i_start, 0)
    pltpu.sync_copy(zeros_hbm.at[pl.ds(row0, RS)], acc.at[pl.ds(row0, RS)])

    @pl.when(s == 0)
    def _():
        pltpu.sync_copy(zeros_hbm.at[pl.ds(NS * RS, NT)],
                        acc.at[pl.ds(NS * RS, NT)])

    pltpu.sync_copy(ones_hbm, ones_v)
    lax.fori_loop(0, kcnt, i_drain, 0)
    plsc.subcore_barrier()

    def s_start(k):
        pltpu.async_copy(ones_v, acc.at[didx.at[k]], ssem, add=True)

    def s_wait(k):
        pltpu.make_async_copy(ones_v, acc.at[didx.at[k]], ssem).wait()

    def body(k, carry):
        s_start(k)

        @pl.when(k >= DDEPTH)
        def _():
            s_wait(k - DDEPTH)
        return carry

    lax.fori_loop(0, kcnt, body, 0)
    lax.fori_loop(kcnt - DDEPTH, kcnt, lambda k, cy: (s_wait(k), cy)[1], 0)
    plsc.subcore_barrier()
    pltpu.sync_copy(acc.at[pl.ds(row0, RS)],
                    out_hbm.at[pl.ds(c * N + row0, RS)])

    @pl.when(s == 0)
    def _():
        pltpu.sync_copy(acc.at[pl.ds(NS * RS, NT)],
                        out_hbm.at[pl.ds(c * N + NS * RS, NT)])


@functools.partial(
    pl.kernel,
    out_type=jax.ShapeDtypeStruct((NC * N, D), jnp.float32),
    mesh=_mesh,
    scratch_types=[
        pltpu.VMEM((6, CH), jnp.int32),
        pltpu.VMEM((3, CH), jnp.int32),
        pltpu.VMEM((6, CH, D), jnp.float32),
        pltpu.VMEM_SHARED((N, D), jnp.float32),
        pltpu.SemaphoreType.DMA,
        pltpu.SemaphoreType.DMA,
        pltpu.SemaphoreType.DMA,
        pltpu.SemaphoreType.DMA,
    ],
)
def _agg_kernel(hws_hbm, src_hbm, dst_hbm, zeros_hbm, out_hbm,
                sidx, didx, rows, acc, gsem, ssem, issem, idsem):
    c = lax.axis_index("c")
    s = lax.axis_index("s")
    w = c * NS + s
    k0 = w * KPW
    row0 = s * RS
    pltpu.sync_copy(zeros_hbm.at[pl.ds(row0, RS)], acc.at[pl.ds(row0, RS)])

    @pl.when(s == 0)
    def _():
        pltpu.sync_copy(zeros_hbm.at[pl.ds(NS * RS, NT)],
                        acc.at[pl.ds(NS * RS, NT)])

    plsc.subcore_barrier()
    klast = jnp.where(w < XTRA, KPW, KPW - 1)

    def erow(k):
        return jnp.where(k < KPW, k0 + k, NW * KPW + w)

    def is_start(k, j):
        pltpu.async_copy(src_hbm.at[erow(k)], sidx.at[j], issem)

    def is_wait(k, j):
        pltpu.make_async_copy(src_hbm.at[erow(k)], sidx.at[j], issem).wait()

    def id_start(k, j):
        pltpu.async_copy(dst_hbm.at[erow(k)], didx.at[j], idsem)

    def id_wait(k, j):
        pltpu.make_async_copy(dst_hbm.at[erow(k)], didx.at[j], idsem).wait()

    def g_start(j, b):
        pltpu.async_copy(hws_hbm.at[sidx.at[j]], rows.at[b], gsem)

    def g_wait(j, b):
        pltpu.make_async_copy(hws_hbm.at[sidx.at[j]], rows.at[b], gsem).wait()

    def s_start(j, b):
        pltpu.async_copy(rows.at[b], acc.at[didx.at[j]], ssem, add=True)

    def s_wait(j, b):
        pltpu.make_async_copy(rows.at[b], acc.at[didx.at[j]], ssem).wait()

    pltpu.sync_copy(src_hbm.at[erow(0)], sidx.at[0])
    pltpu.sync_copy(src_hbm.at[erow(1)], sidx.at[1])
    pltpu.sync_copy(src_hbm.at[erow(2)], sidx.at[2])
    is_start(3, 3)
    is_start(4, 4)
    id_start(0, 0)
    id_start(1, 1)
    g_start(0, 0)
    g_start(1, 1)
    g_start(2, 2)

    def step(t, u):
        k = 12 * t + u
        b = u % 6
        j3 = u % 3
        if u == 0:
            @pl.when(t > 0)
            def _():
                s_wait(2, 5)
        else:
            s_wait((u - 1) % 3, (u - 1) % 6)

        @pl.when(k + 3 <= klast)
        def _():
            is_wait(k + 3, (u + 3) % 6)
            g_start((u + 3) % 6, (u + 3) % 6)

        @pl.when(k + 5 <= klast)
        def _():
            is_start(k + 5, (u + 5) % 6)

        @pl.when(k + 2 <= klast)
        def _():
            id_start(k + 2, (u + 2) % 3)
        g_wait(b, b)
        id_wait(k, j3)
        s_start(j3, b)

    def body(t, carry):
        for u in range(12):
            step(t, u)
        return carry

    lax.fori_loop(0, KPW // 12, body, 0)

    @pl.when(w < XTRA)
    def _():
        g_wait(0, 0)
        id_wait(KPW, 0)
        s_start(0, 0)
        s_wait(0, 0)

    s_wait((KPW - 1) % 3, (KPW - 1) % 6)

    plsc.subcore_barrier()
    pltpu.sync_copy(acc.at[pl.ds(row0, RS)],
                    out_hbm.at[pl.ds(c * N + row0, RS)])

    @pl.when(s == 0)
    def _():
        pltpu.sync_copy(acc.at[pl.ds(NS * RS, NT)],
                        out_hbm.at[pl.ds(c * N + NS * RS, NT)])


R = 1000


def _pro_body(x_ref, ew_ref, eb_ref, w0_ref, degp_ref, h_ref, hws_ref, dis_ref):
    deg = degp_ref[0, :, 0:1] + degp_ref[1, :, 0:1] + 1.0
    dis = lax.rsqrt(deg)
    h = jnp.dot(x_ref[...], ew_ref[...], preferred_element_type=jnp.float32)
    h = h + eb_ref[...]
    hw = jnp.dot(h, w0_ref[...], preferred_element_type=jnp.float32)
    h_ref[...] = h
    hws_ref[...] = hw * dis
    dis_ref[...] = dis


_pro = pl.pallas_call(
    _pro_body,
    grid=(N // R,),
    in_specs=[
        pl.BlockSpec((R, D), lambda i: (i, 0)),
        pl.BlockSpec((D, D), lambda i: (0, 0)),
        pl.BlockSpec((1, D), lambda i: (0, 0)),
        pl.BlockSpec((D, D), lambda i: (0, 0)),
        pl.BlockSpec((NC, R, 16), lambda i: (0, i, 0)),
    ],
    out_specs=[
        pl.BlockSpec((R, D), lambda i: (i, 0)),
        pl.BlockSpec((R, D), lambda i: (i, 0)),
        pl.BlockSpec((R, 1), lambda i: (i, 0)),
    ],
    out_shape=[
        jax.ShapeDtypeStruct((N, D), jnp.float32),
        jax.ShapeDtypeStruct((N, D), jnp.float32),
        jax.ShapeDtypeStruct((N, 1), jnp.float32),
    ],
)


def _ln_update(h_ref, hws_ref, aggp_ref, dis_ref, cb_ref, g_ref, b_ref):
    dis = dis_ref[...]
    agg0 = aggp_ref[0, :, :] + aggp_ref[1, :, :]
    t = (agg0 + hws_ref[...]) * dis + cb_ref[...]
    xn = jnp.maximum(t, 0.0)
    hh = h_ref[...] + xn
    mu = jnp.mean(hh, axis=-1, keepdims=True)
    var = jnp.mean((hh - mu) ** 2, axis=-1, keepdims=True)
    return (hh - mu) * lax.rsqrt(var + 1e-5) * g_ref[...] + b_ref[...]


def _post_body(h_ref, hws_ref, aggp_ref, dis_ref, cb_ref, g_ref, b_ref, wn_ref,
               hn_ref, hwsn_ref):
    hn = _ln_update(h_ref, hws_ref, aggp_ref, dis_ref, cb_ref, g_ref, b_ref)
    hn_ref[...] = hn
    hw = jnp.dot(hn, wn_ref[...], preferred_element_type=jnp.float32)
    hwsn_ref[...] = hw * dis_ref[...]


_post = pl.pallas_call(
    _post_body,
    grid=(N // R,),
    in_specs=[
        pl.BlockSpec((R, D), lambda i: (i, 0)),
        pl.BlockSpec((R, D), lambda i: (i, 0)),
        pl.BlockSpec((NC, R, D), lambda i: (0, i, 0)),
        pl.BlockSpec((R, 1), lambda i: (i, 0)),
        pl.BlockSpec((1, D), lambda i: (0, 0)),
        pl.BlockSpec((1, D), lambda i: (0, 0)),
        pl.BlockSpec((1, D), lambda i: (0, 0)),
        pl.BlockSpec((D, D), lambda i: (0, 0)),
    ],
    out_specs=[
        pl.BlockSpec((R, D), lambda i: (i, 0)),
        pl.BlockSpec((R, D), lambda i: (i, 0)),
    ],
    out_shape=[
        jax.ShapeDtypeStruct((N, D), jnp.float32),
        jax.ShapeDtypeStruct((N, D), jnp.float32),
    ],
)


def _fin_body(h_ref, hws_ref, aggp_ref, dis_ref, cb_ref, g_ref, b_ref, ow_ref,
              batch_ref, out_ref, sums, cnts):
    i = pl.program_id(0)
    hn = _ln_update(h_ref, hws_ref, aggp_ref, dis_ref, cb_ref, g_ref, b_ref)

    @pl.when(i == 0)
    def _():
        sums[...] = jnp.zeros_like(sums)
        cnts[...] = jnp.zeros_like(cnts)

    bvec = batch_ref[0, 0, :]
    onehot = (lax.broadcasted_iota(jnp.int32, (G, R), 0)
              == bvec[None, :]).astype(jnp.float32)
    sums[...] += jnp.dot(onehot, hn, preferred_element_type=jnp.float32)
    cnts[...] += jnp.sum(onehot, axis=1, keepdims=True)

    @pl.when(i == pl.num_programs(0) - 1)
    def _():
        out_ref[...] = (jnp.dot(sums[...], ow_ref[...],
                                preferred_element_type=jnp.float32)
                        / jnp.maximum(cnts[...], 1.0))


_fin = pl.pallas_call(
    _fin_body,
    grid=(N // R,),
    in_specs=[
        pl.BlockSpec((R, D), lambda i: (i, 0)),
        pl.BlockSpec((R, D), lambda i: (i, 0)),
        pl.BlockSpec((NC, R, D), lambda i: (0, i, 0)),
        pl.BlockSpec((R, 1), lambda i: (i, 0)),
        pl.BlockSpec((1, D), lambda i: (0, 0)),
        pl.BlockSpec((1, D), lambda i: (0, 0)),
        pl.BlockSpec((1, D), lambda i: (0, 0)),
        pl.BlockSpec((D, 1), lambda i: (0, 0)),
        pl.BlockSpec((1, 1, R), lambda i: (i, 0, 0)),
    ],
    out_specs=pl.BlockSpec((G, 1), lambda i: (0, 0)),
    out_shape=jax.ShapeDtypeStruct((G, 1), jnp.float32),
    scratch_shapes=[
        pltpu.VMEM((G, D), jnp.float32),
        pltpu.VMEM((G, 1), jnp.float32),
    ],
)


def kernel(x, edge_index, batch, emb_W, emb_b, conv_W, conv_b, ln_g, ln_b, out_W):
    src = edge_index[0].reshape(EROWS, CH)
    dst = edge_index[1].reshape(EROWS, CH)
    dst_d = edge_index[1].reshape(DROWS, DCH)
    zeros_nd = jnp.zeros((N, D), jnp.float32)
    zeros_16 = jnp.zeros((N, 16), jnp.float32)
    ones_16 = jnp.ones((DCH, 16), jnp.float32)

    degp = _deg_kernel(dst_d, zeros_16, ones_16).reshape(NC, N, 16)
    h, hws, dis = _pro(x, emb_W, emb_b.reshape(1, D), conv_W[0], degp)

    out = None
    for i in range(3):
        aggp = _agg_kernel(hws, src, dst, zeros_nd).reshape(NC, N, D)
        cb = conv_b[i].reshape(1, D)
        g = ln_g[i].reshape(1, D)
        b = ln_b[i].reshape(1, D)
        if i < 2:
            h, hws = _post(h, hws, aggp, dis, cb, g, b, conv_W[i + 1])
        else:
            out = _fin(h, hws, aggp, dis, cb, g, b, out_W,
                       batch.reshape(N // R, 1, R))
    return out
